# Initial kernel scaffold; baseline (speedup 1.0000x reference)
import jax
import jax.numpy as jnp
from jax.experimental import pallas as pl


def kernel(node_embed, batch, component, W1, b1, W2, b2):
    raise NotImplementedError("write your pallas kernel here")


# speedup vs baseline: 4.8942x; 4.8942x over previous
import dataclasses
import functools

import jax
import jax.numpy as jnp
from jax import lax
from jax.experimental import pallas as pl
from jax.experimental.pallas import tpu as pltpu
from jax.experimental.pallas import tpu_sc as plsc

N = 50000
D = 256
B = 256
C_MAX = 32
CHUNK = 128
NCHUNKS = (N + CHUNK - 1) // CHUNK
NBUCKET = B * C_MAX
HALF = NBUCKET // 2
NSUB = 16
NWORK = 2 * NSUB
ROWS_PAD = NBUCKET + NWORK
ZROWS = NBUCKET // NWORK
KMAX = -(-NCHUNKS // NSUB)
NPAD = 49 * 1024


def _sc_body(node_hbm, batch_hbm, comp_hbm, zeros_hbm,
             e_out, nbuf, bbuf, cbuf, ibuf, acc):
    cid = lax.axis_index("c")
    sid = lax.axis_index("s")
    base = cid * HALF
    glo = cid * (B // 2)

    pltpu.sync_copy(zeros_hbm, acc.at[pl.ds(sid * ZROWS, ZROWS)])
    plsc.subcore_barrier()

    iota = lax.broadcasted_iota(jnp.int32, (16,), 0)
    dummy = HALF + sid

    @pl.loop(0, KMAX)
    def _(k):
        j = k * NSUB + sid

        @pl.when(j < NCHUNKS)
        def _():
            start = jnp.minimum(j * CHUNK, N - CHUNK)
            pltpu.sync_copy(batch_hbm.at[pl.ds(start, CHUNK)], bbuf)
            pltpu.sync_copy(comp_hbm.at[pl.ds(start, CHUNK)], cbuf)
            for g in range(CHUNK // 16):
                b = bbuf[pl.ds(g * 16, 16)]
                c = cbuf[pl.ds(g * 16, 16)]
                sval = b * C_MAX + c - base
                pos = start + g * 16 + iota
                ok = (sval >= 0) & (sval < HALF) & (pos >= j * CHUNK)
                ibuf[pl.ds(g * 16, 16)] = jnp.where(ok, sval, dummy)
            lo = jnp.min(bbuf[pl.ds(0, 16)])
            hi = jnp.max(bbuf[pl.ds(CHUNK - 16, 16)])
            relevant = (hi >= glo) & (lo < glo + B // 2)

            @pl.when(relevant)
            def _():
                pltpu.sync_copy(node_hbm.at[pl.ds(start, CHUNK)], nbuf)
                pltpu.sync_copy(nbuf, acc.at[ibuf], add=True)

    plsc.subcore_barrier()
    row = sid * ZROWS
    pltpu.sync_copy(acc.at[pl.ds(row, ZROWS)],
                    e_out.at[pl.ds(base + row, ZROWS)])


_sc_compiler_params = pltpu.CompilerParams()
if "needs_layout_passes" in pltpu.CompilerParams.__dataclass_fields__:
    _sc_compiler_params = dataclasses.replace(
        _sc_compiler_params, needs_layout_passes=False)
if "use_tc_tiling_on_sc" in pltpu.CompilerParams.__dataclass_fields__:
    _sc_compiler_params = dataclasses.replace(
        _sc_compiler_params, use_tc_tiling_on_sc=False)

_sc_segment_sum = functools.partial(
    pl.kernel,
    compiler_params=_sc_compiler_params,
    out_type=jax.ShapeDtypeStruct((NBUCKET, D), jnp.float32),
    mesh=plsc.VectorSubcoreMesh(core_axis_name="c", subcore_axis_name="s"),
    scratch_types=[
        pltpu.VMEM((CHUNK, D), jnp.float32),
        pltpu.VMEM((CHUNK,), jnp.int32),
        pltpu.VMEM((CHUNK,), jnp.int32),
        pltpu.VMEM((CHUNK,), jnp.int32),
        pltpu.VMEM_SHARED((HALF + NSUB, D), jnp.float32),
    ],
)(_sc_body)


NC_BLK = 1024


def _numc_body(b_ref, c_ref, o_ref, mx_ref):
    i = pl.program_id(0)

    @pl.when(i == 0)
    def _():
        mx_ref[...] = jnp.full((1, B), -1, jnp.int32)

    giota = lax.broadcasted_iota(jnp.int32, (1, B), 1)
    cand = jnp.where(b_ref[...] == giota, c_ref[...], -1)
    mx_ref[...] = jnp.maximum(mx_ref[...],
                              jnp.max(cand, axis=0, keepdims=True))

    @pl.when(i == NPAD // NC_BLK - 1)
    def _():
        o_ref[...] = (mx_ref[...] + 1).astype(jnp.float32)


def _mlp_body(e_ref, m_ref, w1_ref, b1_ref, w2_ref, b2_ref, o_ref):
    e = e_ref[...]
    h = jnp.dot(e, w1_ref[...], preferred_element_type=jnp.float32)
    h = h + b1_ref[...]
    h = jnp.where(h >= 0, h, 0.01 * h)
    val = jnp.sum(h * w2_ref[...], axis=1, keepdims=True) + b2_ref[0, 0]
    valm = val.reshape(C_MAX, C_MAX)
    ciota = lax.broadcasted_iota(jnp.int32, (1, C_MAX), 1).astype(jnp.float32)
    msk = (ciota < m_ref[...]).astype(jnp.float32)
    v = jnp.sum(valm * msk, axis=1, keepdims=True)
    o_ref[...] = v


def kernel(node_embed, batch, component, W1, b1, W2, b2):
    zeros = jnp.zeros((ZROWS, D), jnp.float32)
    e = _sc_segment_sum(node_embed, batch, component, zeros)

    bpad = jnp.full((NPAD - N,), B, jnp.int32)
    bcol = jnp.concatenate([batch, bpad]).reshape(NPAD, 1)
    ccol = jnp.concatenate(
        [component, jnp.zeros((NPAD - N,), jnp.int32)]
    ).reshape(NPAD, 1)
    numc = pl.pallas_call(
        _numc_body,
        grid=(NPAD // NC_BLK,),
        in_specs=[
            pl.BlockSpec((NC_BLK, 1), lambda i: (i, 0)),
            pl.BlockSpec((NC_BLK, 1), lambda i: (i, 0)),
        ],
        out_specs=pl.BlockSpec((1, B), lambda i: (0, 0)),
        out_shape=jax.ShapeDtypeStruct((1, B), jnp.float32),
        scratch_shapes=[pltpu.VMEM((1, B), jnp.int32)],
    )(bcol, ccol)
    numc = numc.reshape(B, 1)

    rows = C_MAX * C_MAX
    v = pl.pallas_call(
        _mlp_body,
        grid=(NBUCKET // rows,),
        in_specs=[
            pl.BlockSpec((rows, D), lambda i: (i, 0)),
            pl.BlockSpec((C_MAX, 1), lambda i: (i, 0)),
            pl.BlockSpec((D, D), lambda i: (0, 0)),
            pl.BlockSpec((1, D), lambda i: (0, 0)),
            pl.BlockSpec((1, D), lambda i: (0, 0)),
            pl.BlockSpec((1, 1), lambda i: (0, 0)),
        ],
        out_specs=pl.BlockSpec((C_MAX, 1), lambda i: (i, 0)),
        out_shape=jax.ShapeDtypeStruct((B, 1), jnp.float32),
    )(e, numc, W1, b1.reshape(1, D), W2.reshape(1, D), b2.reshape(1, 1))
    return v


# speedup vs baseline: 5.6127x; 1.1468x over previous
import dataclasses
import functools

import jax
import jax.numpy as jnp
from jax import lax
from jax.experimental import pallas as pl
from jax.experimental.pallas import tpu as pltpu
from jax.experimental.pallas import tpu_sc as plsc

N = 50000
D = 256
B = 256
C_MAX = 32
CHUNK = 96
NCHUNKS = (N + CHUNK - 1) // CHUNK
NBUCKET = B * C_MAX
HALF = NBUCKET // 2
NSUB = 16
NWORK = 2 * NSUB
ROWS_PAD = NBUCKET + NWORK
ZROWS = NBUCKET // NWORK
KMAX = -(-NCHUNKS // NSUB)
NPAD = 49 * 1024


def _sc_body(node_hbm, batch_hbm, comp_hbm, zeros_hbm, e_out,
             nbuf, bbufs, cbufs, ibufs, flags, acc,
             semz, semh, semn0, semn1, sems0, sems1):
    cid = lax.axis_index("c")
    sid = lax.axis_index("s")
    base = cid * HALF
    glo = cid * (B // 2)
    semn = (semn0, semn1)
    sems = (sems0, sems1)

    def chunk_start(k):
        j = k * NSUB + sid
        return jnp.minimum(j * CHUNK, N - CHUNK)

    zh = pltpu.make_async_copy(zeros_hbm, acc.at[pl.ds(sid * ZROWS, ZROWS)],
                               semz)
    zh.start()
    hs = []
    for k in range(KMAX):
        start = chunk_start(k)
        hs.append(pltpu.make_async_copy(batch_hbm.at[pl.ds(start, CHUNK)],
                                        bbufs.at[k], semh))
        hs.append(pltpu.make_async_copy(comp_hbm.at[pl.ds(start, CHUNK)],
                                        cbufs.at[k], semh))
    for h in hs:
        h.start()
    for h in hs:
        h.wait()

    iota = lax.broadcasted_iota(jnp.int32, (16,), 0)
    dummy = HALF + sid

    for k in range(KMAX):
        j = k * NSUB + sid
        start = chunk_start(k)
        for g in range(CHUNK // 16):
            b = bbufs[k, pl.ds(g * 16, 16)]
            c = cbufs[k, pl.ds(g * 16, 16)]
            sval = b * C_MAX + c - base
            pos = start + g * 16 + iota
            ok = (sval >= 0) & (sval < HALF) & (pos >= j * CHUNK)
            ibufs[k, pl.ds(g * 16, 16)] = jnp.where(ok, sval, dummy)
        lo = jnp.min(bbufs[k, pl.ds(0, 16)])
        hi = jnp.max(bbufs[k, pl.ds(CHUNK - 16, 16)])
        rel = (hi >= glo) & (lo < glo + B // 2) & (j < NCHUNKS)
        flags[k] = rel.astype(jnp.int32)

    zh.wait()
    plsc.subcore_barrier()

    def node_dma(k, buf):
        return pltpu.make_async_copy(
            node_hbm.at[pl.ds(chunk_start(k), CHUNK)], nbuf.at[buf],
            semn[buf])

    def scat_dma(k, buf):
        return pltpu.make_async_copy(nbuf.at[buf], acc.at[ibufs.at[k]],
                                     sems[buf])

    for k in range(KMAX + 2):
        if k >= 2:
            @pl.when(flags[k - 2] == 1)
            def _(k=k):
                scat_dma(k - 2, (k - 2) % 2).wait()
        if k < KMAX:
            @pl.when(flags[k] == 1)
            def _(k=k):
                node_dma(k, k % 2).start()
        if 1 <= k <= KMAX:
            @pl.when(flags[k - 1] == 1)
            def _(k=k):
                node_dma(k - 1, (k - 1) % 2).wait()
                scat_dma(k - 1, (k - 1) % 2).start(add=True)

    plsc.subcore_barrier()
    row = sid * ZROWS
    pltpu.sync_copy(acc.at[pl.ds(row, ZROWS)],
                    e_out.at[pl.ds(base + row, ZROWS)])


_sc_compiler_params = pltpu.CompilerParams()
if "needs_layout_passes" in pltpu.CompilerParams.__dataclass_fields__:
    _sc_compiler_params = dataclasses.replace(
        _sc_compiler_params, needs_layout_passes=False)
if "use_tc_tiling_on_sc" in pltpu.CompilerParams.__dataclass_fields__:
    _sc_compiler_params = dataclasses.replace(
        _sc_compiler_params, use_tc_tiling_on_sc=False)

_sc_segment_sum = functools.partial(
    pl.kernel,
    compiler_params=_sc_compiler_params,
    out_type=jax.ShapeDtypeStruct((NBUCKET, D), jnp.float32),
    mesh=plsc.VectorSubcoreMesh(core_axis_name="c", subcore_axis_name="s"),
    scratch_types=[
        pltpu.VMEM((2, CHUNK, D), jnp.float32),
        pltpu.VMEM((KMAX, CHUNK), jnp.int32),
        pltpu.VMEM((KMAX, CHUNK), jnp.int32),
        pltpu.VMEM((KMAX, CHUNK), jnp.int32),
        pltpu.SMEM((KMAX,), jnp.int32),
        pltpu.VMEM_SHARED((HALF + NSUB, D), jnp.float32),
        pltpu.SemaphoreType.DMA,
        pltpu.SemaphoreType.DMA,
        pltpu.SemaphoreType.DMA,
        pltpu.SemaphoreType.DMA,
        pltpu.SemaphoreType.DMA,
        pltpu.SemaphoreType.DMA,
    ],
)(_sc_body)


NC_BLK = 1024


def _numc_body(b_ref, c_ref, o_ref, mx_ref):
    i = pl.program_id(0)

    @pl.when(i == 0)
    def _():
        mx_ref[...] = jnp.full((1, B), -1, jnp.int32)

    giota = lax.broadcasted_iota(jnp.int32, (1, B), 1)
    cand = jnp.where(b_ref[...] == giota, c_ref[...], -1)
    mx_ref[...] = jnp.maximum(mx_ref[...],
                              jnp.max(cand, axis=0, keepdims=True))

    @pl.when(i == NPAD // NC_BLK - 1)
    def _():
        o_ref[...] = (mx_ref[...] + 1).astype(jnp.float32)


def _mlp_body(e_ref, m_ref, w1_ref, b1_ref, w2_ref, b2_ref, o_ref):
    e = e_ref[...]
    h = jnp.dot(e, w1_ref[...], preferred_element_type=jnp.float32)
    h = h + b1_ref[...]
    h = jnp.where(h >= 0, h, 0.01 * h)
    val = jnp.sum(h * w2_ref[...], axis=1, keepdims=True) + b2_ref[0, 0]
    valm = val.reshape(C_MAX, C_MAX)
    ciota = lax.broadcasted_iota(jnp.int32, (1, C_MAX), 1).astype(jnp.float32)
    msk = (ciota < m_ref[...]).astype(jnp.float32)
    v = jnp.sum(valm * msk, axis=1, keepdims=True)
    o_ref[...] = v


def kernel(node_embed, batch, component, W1, b1, W2, b2):
    zeros = jnp.zeros((ZROWS, D), jnp.float32)
    e = _sc_segment_sum(node_embed, batch, component, zeros)

    bpad = jnp.full((NPAD - N,), B, jnp.int32)
    bcol = jnp.concatenate([batch, bpad]).reshape(NPAD, 1)
    ccol = jnp.concatenate(
        [component, jnp.zeros((NPAD - N,), jnp.int32)]
    ).reshape(NPAD, 1)
    numc = pl.pallas_call(
        _numc_body,
        grid=(NPAD // NC_BLK,),
        in_specs=[
            pl.BlockSpec((NC_BLK, 1), lambda i: (i, 0)),
            pl.BlockSpec((NC_BLK, 1), lambda i: (i, 0)),
        ],
        out_specs=pl.BlockSpec((1, B), lambda i: (0, 0)),
        out_shape=jax.ShapeDtypeStruct((1, B), jnp.float32),
        scratch_shapes=[pltpu.VMEM((1, B), jnp.int32)],
    )(bcol, ccol)
    numc = numc.reshape(B, 1)

    rows = C_MAX * C_MAX
    v = pl.pallas_call(
        _mlp_body,
        grid=(NBUCKET // rows,),
        in_specs=[
            pl.BlockSpec((rows, D), lambda i: (i, 0)),
            pl.BlockSpec((C_MAX, 1), lambda i: (i, 0)),
            pl.BlockSpec((D, D), lambda i: (0, 0)),
            pl.BlockSpec((1, D), lambda i: (0, 0)),
            pl.BlockSpec((1, D), lambda i: (0, 0)),
            pl.BlockSpec((1, 1), lambda i: (0, 0)),
        ],
        out_specs=pl.BlockSpec((C_MAX, 1), lambda i: (i, 0)),
        out_shape=jax.ShapeDtypeStruct((B, 1), jnp.float32),
    )(e, numc, W1, b1.reshape(1, D), W2.reshape(1, D), b2.reshape(1, 1))
    return v


# speedup vs baseline: 8.4609x; 1.5075x over previous
import dataclasses
import functools

import jax
import jax.numpy as jnp
from jax import lax
from jax.experimental import pallas as pl
from jax.experimental.pallas import tpu as pltpu
from jax.experimental.pallas import tpu_sc as plsc

N = 50000
D = 256
B = 256
C_MAX = 32
CHUNK = 112
NCHUNKS = (N + CHUNK - 1) // CHUNK
NBUCKET = B * C_MAX
HALF = NBUCKET // 2
NSUB = 16
NWORK = 2 * NSUB
ROWS_PAD = NBUCKET + NWORK
ZROWS = NBUCKET // NWORK
KMAX = -(-NCHUNKS // NSUB)
NPAD = 49 * 1024


def _sc_body(node_hbm, batch_hbm, comp_hbm, zeros_hbm, e_out,
             nlo, nhi, bbufs, cbufs, ibufs, flags, acc_lo, acc_hi,
             semz, semh, semn0, semn1, sems0, sems1):
    cid = lax.axis_index("c")
    sid = lax.axis_index("s")
    base = cid * HALF
    glo = cid * (B // 2)
    semn = (semn0, semn1)
    sems = (sems0, sems1)

    def chunk_start(k):
        j = k * NSUB + sid
        return jnp.minimum(j * CHUNK, N - CHUNK)

    def header_dma(k):
        start = chunk_start(k)
        slot = k % 4
        return (pltpu.make_async_copy(batch_hbm.at[pl.ds(start, CHUNK)],
                                      bbufs.at[slot], semh),
                pltpu.make_async_copy(comp_hbm.at[pl.ds(start, CHUNK)],
                                      cbufs.at[slot], semh))

    zl = pltpu.make_async_copy(zeros_hbm, acc_lo.at[pl.ds(sid * ZROWS, ZROWS)],
                               semz)
    zh = pltpu.make_async_copy(zeros_hbm, acc_hi.at[pl.ds(sid * ZROWS, ZROWS)],
                               semz)
    zl.start()
    zh.start()
    for k in range(min(2, KMAX)):
        for h in header_dma(k):
            h.start()

    iota = lax.broadcasted_iota(jnp.int32, (16,), 0)
    dummy = HALF + sid
    HD = D // 2

    zl.wait()
    zh.wait()
    plsc.subcore_barrier()

    def node_dma(k, buf, half):
        src = node_hbm.at[pl.ds(chunk_start(k), CHUNK), pl.ds(half * HD, HD)]
        return pltpu.make_async_copy(src, (nlo, nhi)[half].at[buf],
                                     semn[buf])

    def scat_dma(k, buf, half):
        return pltpu.make_async_copy(
            (nlo, nhi)[half].at[buf],
            (acc_lo, acc_hi)[half].at[ibufs.at[k % 4]], sems[buf])

    for k in range(KMAX + 2):
        if k >= 2:
            @pl.when(flags[k - 2] == 1)
            def _(k=k):
                scat_dma(k - 2, (k - 2) % 2, 0).wait()
                scat_dma(k - 2, (k - 2) % 2, 1).wait()
        if k + 2 < KMAX:
            for h in header_dma(k + 2):
                h.start()
        if k < KMAX:
            j = k * NSUB + sid
            start = chunk_start(k)
            slot = k % 4
            for h in header_dma(k):
                h.wait()
            lo = jnp.min(bbufs[slot, pl.ds(0, 16)])
            hi = jnp.max(bbufs[slot, pl.ds(CHUNK - 16, 16)])
            rel = (hi >= glo) & (lo < glo + B // 2) & (j < NCHUNKS)
            flags[k] = rel.astype(jnp.int32)

            @pl.when(flags[k] == 1)
            def _(k=k, j=j, start=start, slot=slot):
                node_dma(k, k % 2, 0).start()
                node_dma(k, k % 2, 1).start()
                for g in range(CHUNK // 16):
                    b = bbufs[slot, pl.ds(g * 16, 16)]
                    c = cbufs[slot, pl.ds(g * 16, 16)]
                    sval = b * C_MAX + c - base
                    pos = start + g * 16 + iota
                    ok = (sval >= 0) & (sval < HALF) & (pos >= j * CHUNK)
                    ibufs[slot, pl.ds(g * 16, 16)] = jnp.where(ok, sval, dummy)
        if 1 <= k <= KMAX:
            @pl.when(flags[k - 1] == 1)
            def _(k=k):
                node_dma(k - 1, (k - 1) % 2, 0).wait()
                node_dma(k - 1, (k - 1) % 2, 1).wait()
                scat_dma(k - 1, (k - 1) % 2, 0).start(add=True)
                scat_dma(k - 1, (k - 1) % 2, 1).start(add=True)

    plsc.subcore_barrier()
    row = sid * ZROWS
    pltpu.sync_copy(acc_lo.at[pl.ds(row, ZROWS)],
                    e_out.at[pl.ds(base + row, ZROWS), pl.ds(0, HD)])
    pltpu.sync_copy(acc_hi.at[pl.ds(row, ZROWS)],
                    e_out.at[pl.ds(base + row, ZROWS), pl.ds(HD, HD)])


_sc_compiler_params = pltpu.CompilerParams()
if "needs_layout_passes" in pltpu.CompilerParams.__dataclass_fields__:
    _sc_compiler_params = dataclasses.replace(
        _sc_compiler_params, needs_layout_passes=False)

_sc_segment_sum = functools.partial(
    pl.kernel,
    compiler_params=_sc_compiler_params,
    out_type=jax.ShapeDtypeStruct((NBUCKET, D), jnp.float32),
    mesh=plsc.VectorSubcoreMesh(core_axis_name="c", subcore_axis_name="s"),
    scratch_types=[
        pltpu.VMEM((2, CHUNK, D // 2), jnp.float32),
        pltpu.VMEM((2, CHUNK, D // 2), jnp.float32),
        pltpu.VMEM((4, CHUNK), jnp.int32),
        pltpu.VMEM((4, CHUNK), jnp.int32),
        pltpu.VMEM((4, CHUNK), jnp.int32),
        pltpu.SMEM((KMAX,), jnp.int32),
        pltpu.VMEM_SHARED((HALF + NSUB, D // 2), jnp.float32),
        pltpu.VMEM_SHARED((HALF + NSUB, D // 2), jnp.float32),
        pltpu.SemaphoreType.DMA,
        pltpu.SemaphoreType.DMA,
        pltpu.SemaphoreType.DMA,
        pltpu.SemaphoreType.DMA,
        pltpu.SemaphoreType.DMA,
        pltpu.SemaphoreType.DMA,
    ],
)(_sc_body)


NC_BLK = 1024


def _numc_body(b_ref, c_ref, o_ref, mx_ref):
    i = pl.program_id(0)

    @pl.when(i == 0)
    def _():
        mx_ref[...] = jnp.full((1, B), -1, jnp.int32)

    giota = lax.broadcasted_iota(jnp.int32, (1, B), 1)
    cand = jnp.where(b_ref[...] == giota, c_ref[...], -1)
    mx_ref[...] = jnp.maximum(mx_ref[...],
                              jnp.max(cand, axis=0, keepdims=True))

    @pl.when(i == NPAD // NC_BLK - 1)
    def _():
        o_ref[...] = (mx_ref[...] + 1).astype(jnp.float32)


def _mlp_body(e_ref, m_ref, w1_ref, b1_ref, w2_ref, b2_ref, o_ref):
    e = e_ref[...]
    h = jnp.dot(e, w1_ref[...], preferred_element_type=jnp.float32)
    h = h + b1_ref[...]
    h = jnp.where(h >= 0, h, 0.01 * h)
    val = jnp.sum(h * w2_ref[...], axis=1, keepdims=True) + b2_ref[0, 0]
    valm = val.reshape(C_MAX, C_MAX)
    ciota = lax.broadcasted_iota(jnp.int32, (1, C_MAX), 1).astype(jnp.float32)
    msk = (ciota < m_ref[...]).astype(jnp.float32)
    v = jnp.sum(valm * msk, axis=1, keepdims=True)
    o_ref[...] = v


def kernel(node_embed, batch, component, W1, b1, W2, b2):
    zeros = jnp.zeros((ZROWS, D // 2), jnp.float32)
    e = _sc_segment_sum(node_embed, batch, component, zeros)

    bpad = jnp.full((NPAD - N,), B, jnp.int32)
    bcol = jnp.concatenate([batch, bpad]).reshape(NPAD, 1)
    ccol = jnp.concatenate(
        [component, jnp.zeros((NPAD - N,), jnp.int32)]
    ).reshape(NPAD, 1)
    numc = pl.pallas_call(
        _numc_body,
        grid=(NPAD // NC_BLK,),
        in_specs=[
            pl.BlockSpec((NC_BLK, 1), lambda i: (i, 0)),
            pl.BlockSpec((NC_BLK, 1), lambda i: (i, 0)),
        ],
        out_specs=pl.BlockSpec((1, B), lambda i: (0, 0)),
        out_shape=jax.ShapeDtypeStruct((1, B), jnp.float32),
        scratch_shapes=[pltpu.VMEM((1, B), jnp.int32)],
    )(bcol, ccol)
    numc = numc.reshape(B, 1)

    rows = C_MAX * C_MAX
    v = pl.pallas_call(
        _mlp_body,
        grid=(NBUCKET // rows,),
        in_specs=[
            pl.BlockSpec((rows, D), lambda i: (i, 0)),
            pl.BlockSpec((C_MAX, 1), lambda i: (i, 0)),
            pl.BlockSpec((D, D), lambda i: (0, 0)),
            pl.BlockSpec((1, D), lambda i: (0, 0)),
            pl.BlockSpec((1, D), lambda i: (0, 0)),
            pl.BlockSpec((1, 1), lambda i: (0, 0)),
        ],
        out_specs=pl.BlockSpec((C_MAX, 1), lambda i: (i, 0)),
        out_shape=jax.ShapeDtypeStruct((B, 1), jnp.float32),
    )(e, numc, W1, b1.reshape(1, D), W2.reshape(1, D), b2.reshape(1, 1))
    return v


# speedup vs baseline: 11.0535x; 1.3064x over previous
import dataclasses
import functools

import jax
import jax.numpy as jnp
from jax import lax
from jax.experimental import pallas as pl
from jax.experimental.pallas import tpu as pltpu
from jax.experimental.pallas import tpu_sc as plsc

N = 50000
D = 256
B = 256
C_MAX = 32
CHUNK = 112
NCHUNKS = (N + CHUNK - 1) // CHUNK
NBUCKET = B * C_MAX
HALF = NBUCKET // 2
NSUB = 16
NWORK = 2 * NSUB
ROWS_PAD = NBUCKET + NWORK
ZROWS = NBUCKET // NWORK
KMAX = -(-NCHUNKS // NSUB)
NPAD = 49 * 1024


def _sc_body(node_hbm, batch_hbm, comp_hbm, zeros_hbm, e_out,
             nlo, nhi, bbufs, cbufs, ibufs, flags, acc_lo, acc_hi,
             semz, semh, semn0, semn1, sems0, sems1):
    cid = lax.axis_index("c")
    sid = lax.axis_index("s")
    base = cid * HALF
    glo = cid * (B // 2)
    semn = (semn0, semn1)
    sems = (sems0, sems1)

    def chunk_start(k):
        j = k * NSUB + sid
        return jnp.minimum(j * CHUNK, N - CHUNK)

    def header_dma(k):
        start = chunk_start(k)
        slot = k % 4
        return (pltpu.make_async_copy(batch_hbm.at[pl.ds(start, CHUNK)],
                                      bbufs.at[slot], semh),
                pltpu.make_async_copy(comp_hbm.at[pl.ds(start, CHUNK)],
                                      cbufs.at[slot], semh))

    zl = pltpu.make_async_copy(zeros_hbm, acc_lo.at[pl.ds(sid * ZROWS, ZROWS)],
                               semz)
    zh = pltpu.make_async_copy(zeros_hbm, acc_hi.at[pl.ds(sid * ZROWS, ZROWS)],
                               semz)
    zl.start()
    zh.start()
    for k in range(min(2, KMAX)):
        for h in header_dma(k):
            h.start()

    iota = lax.broadcasted_iota(jnp.int32, (16,), 0)
    dummy = HALF + sid
    HD = D // 2

    zl.wait()
    zh.wait()
    plsc.subcore_barrier()

    def node_dma(k, buf, half):
        src = node_hbm.at[pl.ds(chunk_start(k), CHUNK), pl.ds(half * HD, HD)]
        return pltpu.make_async_copy(src, (nlo, nhi)[half].at[buf],
                                     semn[buf])

    def scat_dma(k, buf, half):
        return pltpu.make_async_copy(
            (nlo, nhi)[half].at[buf],
            (acc_lo, acc_hi)[half].at[ibufs.at[k % 4]], sems[buf])

    for k in range(KMAX + 2):
        if k >= 2:
            @pl.when(flags[k - 2] == 1)
            def _(k=k):
                scat_dma(k - 2, (k - 2) % 2, 0).wait()
                scat_dma(k - 2, (k - 2) % 2, 1).wait()
        if k + 2 < KMAX:
            for h in header_dma(k + 2):
                h.start()
        if k < KMAX:
            j = k * NSUB + sid
            start = chunk_start(k)
            slot = k % 4
            for h in header_dma(k):
                h.wait()
            lo = jnp.min(bbufs[slot, pl.ds(0, 16)])
            hi = jnp.max(bbufs[slot, pl.ds(CHUNK - 16, 16)])
            rel = (hi >= glo) & (lo < glo + B // 2) & (j < NCHUNKS)
            flags[k] = rel.astype(jnp.int32)

            @pl.when(flags[k] == 1)
            def _(k=k, j=j, start=start, slot=slot):
                node_dma(k, k % 2, 0).start()
                node_dma(k, k % 2, 1).start()
                for g in range(CHUNK // 16):
                    b = bbufs[slot, pl.ds(g * 16, 16)]
                    c = cbufs[slot, pl.ds(g * 16, 16)]
                    sval = b * C_MAX + c - base
                    pos = start + g * 16 + iota
                    ok = (sval >= 0) & (sval < HALF) & (pos >= j * CHUNK)
                    ibufs[slot, pl.ds(g * 16, 16)] = jnp.where(ok, sval, dummy)
        if 1 <= k <= KMAX:
            @pl.when(flags[k - 1] == 1)
            def _(k=k):
                node_dma(k - 1, (k - 1) % 2, 0).wait()
                node_dma(k - 1, (k - 1) % 2, 1).wait()
                scat_dma(k - 1, (k - 1) % 2, 0).start(add=True)
                scat_dma(k - 1, (k - 1) % 2, 1).start(add=True)

    plsc.subcore_barrier()
    row = sid * ZROWS
    pltpu.sync_copy(acc_lo.at[pl.ds(row, ZROWS)],
                    e_out.at[pl.ds(base + row, ZROWS), pl.ds(0, HD)])
    pltpu.sync_copy(acc_hi.at[pl.ds(row, ZROWS)],
                    e_out.at[pl.ds(base + row, ZROWS), pl.ds(HD, HD)])


_sc_compiler_params = pltpu.CompilerParams()
if "needs_layout_passes" in pltpu.CompilerParams.__dataclass_fields__:
    _sc_compiler_params = dataclasses.replace(
        _sc_compiler_params, needs_layout_passes=False)

_sc_segment_sum = functools.partial(
    pl.kernel,
    compiler_params=_sc_compiler_params,
    out_type=jax.ShapeDtypeStruct((NBUCKET, D), jnp.float32),
    mesh=plsc.VectorSubcoreMesh(core_axis_name="c", subcore_axis_name="s"),
    scratch_types=[
        pltpu.VMEM((2, CHUNK, D // 2), jnp.float32),
        pltpu.VMEM((2, CHUNK, D // 2), jnp.float32),
        pltpu.VMEM((4, CHUNK), jnp.int32),
        pltpu.VMEM((4, CHUNK), jnp.int32),
        pltpu.VMEM((4, CHUNK), jnp.int32),
        pltpu.SMEM((KMAX,), jnp.int32),
        pltpu.VMEM_SHARED((HALF + NSUB, D // 2), jnp.float32),
        pltpu.VMEM_SHARED((HALF + NSUB, D // 2), jnp.float32),
        pltpu.SemaphoreType.DMA,
        pltpu.SemaphoreType.DMA,
        pltpu.SemaphoreType.DMA,
        pltpu.SemaphoreType.DMA,
        pltpu.SemaphoreType.DMA,
        pltpu.SemaphoreType.DMA,
    ],
)(_sc_body)


NC_ROWS = 8
CHK = 128


def _numc_body(b_ref, c_ref, o_ref, mx_ref):
    i = pl.program_id(0)

    @pl.when(i == 0)
    def _():
        mx_ref[...] = jnp.full((1, B), -1, jnp.int32)

    giota = lax.broadcasted_iota(jnp.int32, (1, B), 1)
    bt = jnp.transpose(b_ref[...])
    ct = jnp.transpose(c_ref[...])
    mx = mx_ref[...]
    for j in range(NC_ROWS):
        cand = jnp.where(bt[:, j:j + 1] == giota,
                         ct[:, j:j + 1], -1)
        mx = jnp.maximum(mx, jnp.max(cand, axis=0, keepdims=True))
    mx_ref[...] = mx

    @pl.when(i == NPAD // (NC_ROWS * CHK) - 1)
    def _():
        o_ref[...] = (mx_ref[...] + 1).astype(jnp.float32)


def _mlp_body(e_ref, m_ref, w1_ref, b1_ref, w2_ref, b2_ref, o_ref):
    e = e_ref[...]
    h = jnp.dot(e, w1_ref[...], preferred_element_type=jnp.float32)
    h = h + b1_ref[...]
    h = jnp.where(h >= 0, h, 0.01 * h)
    val = jnp.sum(h * w2_ref[...], axis=1, keepdims=True) + b2_ref[0, 0]
    valm = val.reshape(C_MAX, C_MAX)
    ciota = lax.broadcasted_iota(jnp.int32, (1, C_MAX), 1).astype(jnp.float32)
    msk = (ciota < m_ref[...]).astype(jnp.float32)
    v = jnp.sum(valm * msk, axis=1, keepdims=True)
    o_ref[...] = v


def kernel(node_embed, batch, component, W1, b1, W2, b2):
    bpad = jnp.full((NPAD - N,), B, jnp.int32)
    b2d = jnp.concatenate([batch, bpad]).reshape(NPAD // CHK, CHK)
    c2d = jnp.concatenate(
        [component, jnp.zeros((NPAD - N,), jnp.int32)]
    ).reshape(NPAD // CHK, CHK)
    numc = pl.pallas_call(
        _numc_body,
        grid=(NPAD // (NC_ROWS * CHK),),
        in_specs=[
            pl.BlockSpec((NC_ROWS, CHK), lambda i: (i, 0)),
            pl.BlockSpec((NC_ROWS, CHK), lambda i: (i, 0)),
        ],
        out_specs=pl.BlockSpec((1, B), lambda i: (0, 0)),
        out_shape=jax.ShapeDtypeStruct((1, B), jnp.float32),
        scratch_shapes=[pltpu.VMEM((1, B), jnp.int32)],
    )(b2d, c2d)
    numc = numc.reshape(B, 1)

    zeros = jnp.zeros((ZROWS, D // 2), jnp.float32)
    e = _sc_segment_sum(node_embed, batch, component, zeros)

    rows = C_MAX * C_MAX
    v = pl.pallas_call(
        _mlp_body,
        grid=(NBUCKET // rows,),
        in_specs=[
            pl.BlockSpec((rows, D), lambda i: (i, 0)),
            pl.BlockSpec((C_MAX, 1), lambda i: (i, 0)),
            pl.BlockSpec((D, D), lambda i: (0, 0)),
            pl.BlockSpec((1, D), lambda i: (0, 0)),
            pl.BlockSpec((1, D), lambda i: (0, 0)),
            pl.BlockSpec((1, 1), lambda i: (0, 0)),
        ],
        out_specs=pl.BlockSpec((C_MAX, 1), lambda i: (i, 0)),
        out_shape=jax.ShapeDtypeStruct((B, 1), jnp.float32),
    )(e, numc, W1, b1.reshape(1, D), W2.reshape(1, D), b2.reshape(1, 1))
    return v


# speedup vs baseline: 11.4296x; 1.0340x over previous
import dataclasses
import functools

import jax
import jax.numpy as jnp
from jax import lax
from jax.experimental import pallas as pl
from jax.experimental.pallas import tpu as pltpu
from jax.experimental.pallas import tpu_sc as plsc

N = 50000
D = 256
B = 256
C_MAX = 32
CHUNK = 112
NCHUNKS = (N + CHUNK - 1) // CHUNK
NBUCKET = B * C_MAX
HALF = NBUCKET // 2
NSUB = 16
NWORK = 2 * NSUB
ROWS_PAD = NBUCKET + NWORK
ZROWS = NBUCKET // NWORK
KMAX = -(-NCHUNKS // NSUB)
NPAD = 49 * 1024


def _sc_body(node_hbm, batch_hbm, comp_hbm, zeros_hbm, e_out,
             nlo, nhi, bbufs, cbufs, ibufs, flags, acc_lo, acc_hi,
             semz, semh, semn0, semn1, sems0, sems1):
    cid = lax.axis_index("c")
    sid = lax.axis_index("s")
    base = cid * HALF
    glo = cid * (B // 2)
    semn = (semn0, semn1)
    sems = (sems0, sems1)

    def chunk_start(k):
        j = k * NSUB + sid
        return jnp.minimum(j * CHUNK, N - CHUNK)

    def header_dma(k):
        start = chunk_start(k)
        slot = k % 4
        return (pltpu.make_async_copy(batch_hbm.at[pl.ds(start, CHUNK)],
                                      bbufs.at[slot], semh),
                pltpu.make_async_copy(comp_hbm.at[pl.ds(start, CHUNK)],
                                      cbufs.at[slot], semh))

    zl = pltpu.make_async_copy(zeros_hbm, acc_lo.at[pl.ds(sid * ZROWS, ZROWS)],
                               semz)
    zh = pltpu.make_async_copy(zeros_hbm, acc_hi.at[pl.ds(sid * ZROWS, ZROWS)],
                               semz)
    zl.start()
    zh.start()
    for k in range(min(2, KMAX)):
        for h in header_dma(k):
            h.start()

    iota = lax.broadcasted_iota(jnp.int32, (16,), 0)
    dummy = HALF + sid
    HD = D // 2

    zl.wait()
    zh.wait()
    plsc.subcore_barrier()

    def node_dma(k, buf, half):
        src = node_hbm.at[pl.ds(chunk_start(k), CHUNK), pl.ds(half * HD, HD)]
        return pltpu.make_async_copy(src, (nlo, nhi)[half].at[buf],
                                     semn[buf])

    def scat_dma(k, buf, half):
        return pltpu.make_async_copy(
            (nlo, nhi)[half].at[buf],
            (acc_lo, acc_hi)[half].at[ibufs.at[k % 4]], sems[buf])

    for k in range(KMAX + 2):
        if k >= 2:
            @pl.when(flags[k - 2] == 1)
            def _(k=k):
                scat_dma(k - 2, (k - 2) % 2, 0).wait()
                scat_dma(k - 2, (k - 2) % 2, 1).wait()
        if k + 2 < KMAX:
            for h in header_dma(k + 2):
                h.start()
        if k < KMAX:
            j = k * NSUB + sid
            start = chunk_start(k)
            slot = k % 4
            for h in header_dma(k):
                h.wait()
            lo = jnp.min(bbufs[slot, pl.ds(0, 16)])
            hi = jnp.max(bbufs[slot, pl.ds(CHUNK - 16, 16)])
            rel = (hi >= glo) & (lo < glo + B // 2) & (j < NCHUNKS)
            flags[k] = rel.astype(jnp.int32)

            @pl.when(flags[k] == 1)
            def _(k=k, j=j, start=start, slot=slot):
                node_dma(k, k % 2, 0).start()
                node_dma(k, k % 2, 1).start()
                for g in range(CHUNK // 16):
                    b = bbufs[slot, pl.ds(g * 16, 16)]
                    c = cbufs[slot, pl.ds(g * 16, 16)]
                    sval = b * C_MAX + c - base
                    pos = start + g * 16 + iota
                    ok = (sval >= 0) & (sval < HALF) & (pos >= j * CHUNK)
                    ibufs[slot, pl.ds(g * 16, 16)] = jnp.where(ok, sval, dummy)
        if 1 <= k <= KMAX:
            @pl.when(flags[k - 1] == 1)
            def _(k=k):
                node_dma(k - 1, (k - 1) % 2, 0).wait()
                node_dma(k - 1, (k - 1) % 2, 1).wait()
                scat_dma(k - 1, (k - 1) % 2, 0).start(add=True)
                scat_dma(k - 1, (k - 1) % 2, 1).start(add=True)

    plsc.subcore_barrier()
    row = sid * ZROWS
    pltpu.sync_copy(acc_lo.at[pl.ds(row, ZROWS)],
                    e_out.at[pl.ds(base + row, ZROWS), pl.ds(0, HD)])
    pltpu.sync_copy(acc_hi.at[pl.ds(row, ZROWS)],
                    e_out.at[pl.ds(base + row, ZROWS), pl.ds(HD, HD)])


_sc_compiler_params = pltpu.CompilerParams()
if "needs_layout_passes" in pltpu.CompilerParams.__dataclass_fields__:
    _sc_compiler_params = dataclasses.replace(
        _sc_compiler_params, needs_layout_passes=False)

_sc_segment_sum = functools.partial(
    pl.kernel,
    compiler_params=_sc_compiler_params,
    out_type=jax.ShapeDtypeStruct((NBUCKET, D), jnp.float32),
    mesh=plsc.VectorSubcoreMesh(core_axis_name="c", subcore_axis_name="s"),
    scratch_types=[
        pltpu.VMEM((2, CHUNK, D // 2), jnp.float32),
        pltpu.VMEM((2, CHUNK, D // 2), jnp.float32),
        pltpu.VMEM((4, CHUNK), jnp.int32),
        pltpu.VMEM((4, CHUNK), jnp.int32),
        pltpu.VMEM((4, CHUNK), jnp.int32),
        pltpu.SMEM((KMAX,), jnp.int32),
        pltpu.VMEM_SHARED((HALF + NSUB, D // 2), jnp.float32),
        pltpu.VMEM_SHARED((HALF + NSUB, D // 2), jnp.float32),
        pltpu.SemaphoreType.DMA,
        pltpu.SemaphoreType.DMA,
        pltpu.SemaphoreType.DMA,
        pltpu.SemaphoreType.DMA,
        pltpu.SemaphoreType.DMA,
        pltpu.SemaphoreType.DMA,
    ],
)(_sc_body)


NC_ROWS = 8
CHK = 128


def _numc_body(b_ref, c_ref, o_ref, mx_ref):
    i = pl.program_id(0)

    @pl.when(i == 0)
    def _():
        mx_ref[...] = jnp.full((1, B), -1, jnp.int32)

    giota = lax.broadcasted_iota(jnp.int32, (1, B), 1)
    bt = jnp.transpose(b_ref[...])
    ct = jnp.transpose(c_ref[...])
    mx = mx_ref[...]
    for j in range(NC_ROWS):
        cand = jnp.where(bt[:, j:j + 1] == giota,
                         ct[:, j:j + 1], -1)
        mx = jnp.maximum(mx, jnp.max(cand, axis=0, keepdims=True))
    mx_ref[...] = mx

    @pl.when(i == NPAD // (NC_ROWS * CHK) - 1)
    def _():
        o_ref[...] = (mx_ref[...] + 1).astype(jnp.float32)


def _mlp_body(e_ref, m_ref, w1_ref, b1_ref, w2_ref, b2_ref, o_ref):
    e = e_ref[...]
    h = jnp.dot(e, w1_ref[...], preferred_element_type=jnp.float32)
    h = h + b1_ref[...]
    h = jnp.where(h >= 0, h, 0.01 * h)
    val = jnp.dot(h, w2_ref[...],
                  preferred_element_type=jnp.float32) + b2_ref[0, 0]
    valm = val.reshape(-1, C_MAX)
    ciota = lax.broadcasted_iota(jnp.int32, (1, C_MAX), 1).astype(jnp.float32)
    msk = (ciota < m_ref[...]).astype(jnp.float32)
    v = jnp.sum(valm * msk, axis=1, keepdims=True)
    o_ref[...] = v


def kernel(node_embed, batch, component, W1, b1, W2, b2):
    bpad = jnp.full((NPAD - N,), B, jnp.int32)
    b2d = jnp.concatenate([batch, bpad]).reshape(NPAD // CHK, CHK)
    c2d = jnp.concatenate(
        [component, jnp.zeros((NPAD - N,), jnp.int32)]
    ).reshape(NPAD // CHK, CHK)
    numc = pl.pallas_call(
        _numc_body,
        grid=(NPAD // (NC_ROWS * CHK),),
        in_specs=[
            pl.BlockSpec((NC_ROWS, CHK), lambda i: (i, 0)),
            pl.BlockSpec((NC_ROWS, CHK), lambda i: (i, 0)),
        ],
        out_specs=pl.BlockSpec((1, B), lambda i: (0, 0)),
        out_shape=jax.ShapeDtypeStruct((1, B), jnp.float32),
        scratch_shapes=[pltpu.VMEM((1, B), jnp.int32)],
    )(b2d, c2d)
    numc = numc.reshape(B, 1)

    zeros = jnp.zeros((ZROWS, D // 2), jnp.float32)
    e = _sc_segment_sum(node_embed, batch, component, zeros)

    rows = 2048
    v = pl.pallas_call(
        _mlp_body,
        grid=(NBUCKET // rows,),
        in_specs=[
            pl.BlockSpec((rows, D), lambda i: (i, 0)),
            pl.BlockSpec((rows // C_MAX, 1), lambda i: (i, 0)),
            pl.BlockSpec((D, D), lambda i: (0, 0)),
            pl.BlockSpec((1, D), lambda i: (0, 0)),
            pl.BlockSpec((D, 1), lambda i: (0, 0)),
            pl.BlockSpec((1, 1), lambda i: (0, 0)),
        ],
        out_specs=pl.BlockSpec((rows // C_MAX, 1), lambda i: (i, 0)),
        out_shape=jax.ShapeDtypeStruct((B, 1), jnp.float32),
    )(e, numc, W1, b1.reshape(1, D), W2, b2.reshape(1, 1))
    return v


# speedup vs baseline: 12.1274x; 1.0611x over previous
import dataclasses
import functools

import jax
import jax.numpy as jnp
from jax import lax
from jax.experimental import pallas as pl
from jax.experimental.pallas import tpu as pltpu
from jax.experimental.pallas import tpu_sc as plsc

N = 50000
D = 256
B = 256
C_MAX = 32
CHUNK = 112
NCHUNKS = (N + CHUNK - 1) // CHUNK
NBUCKET = B * C_MAX
HALF = NBUCKET // 2
NSUB = 16
NWORK = 2 * NSUB
ROWS_PAD = NBUCKET + NWORK
ZROWS = NBUCKET // NWORK
KMAX = -(-NCHUNKS // NSUB)
NPAD = 49 * 1024


def _sc_body(node_hbm, batch_hbm, comp_hbm, zeros_hbm, e_out,
             nlo, nhi, bbufs, cbufs, ibufs, flags, acc_lo, acc_hi,
             semz, semh, semn0, semn1, sems0, sems1):
    cid = lax.axis_index("c")
    sid = lax.axis_index("s")
    base = cid * HALF
    glo = cid * (B // 2)
    semn = (semn0, semn1)
    sems = (sems0, sems1)

    def chunk_start(k):
        j = k * NSUB + sid
        return jnp.minimum(j * CHUNK, N - CHUNK)

    def header_dma(k):
        start = chunk_start(k)
        slot = lax.rem(k, 4) if isinstance(k, jax.Array) else k % 4
        return (pltpu.make_async_copy(batch_hbm.at[pl.ds(start, CHUNK)],
                                      bbufs.at[slot], semh),
                pltpu.make_async_copy(comp_hbm.at[pl.ds(start, CHUNK)],
                                      cbufs.at[slot], semh))

    zl = pltpu.make_async_copy(zeros_hbm, acc_lo.at[pl.ds(sid * ZROWS, ZROWS)],
                               semz)
    zh = pltpu.make_async_copy(zeros_hbm, acc_hi.at[pl.ds(sid * ZROWS, ZROWS)],
                               semz)
    zl.start()
    zh.start()
    for k in range(min(2, KMAX)):
        for h in header_dma(k):
            h.start()

    iota = lax.broadcasted_iota(jnp.int32, (16,), 0)
    dummy = HALF + sid
    HD = D // 2

    zl.wait()
    zh.wait()
    plsc.subcore_barrier()

    def node_dma(k, half):
        buf = lax.rem(k, 2)
        src = node_hbm.at[pl.ds(chunk_start(k), CHUNK), pl.ds(half * HD, HD)]
        return pltpu.make_async_copy(src, (nlo, nhi)[half].at[buf],
                                     semn[half])

    def scat_dma(k, half):
        buf = lax.rem(k, 2)
        return pltpu.make_async_copy(
            (nlo, nhi)[half].at[buf],
            (acc_lo, acc_hi)[half].at[ibufs.at[lax.rem(k, 4)]], sems[half])

    @pl.loop(0, KMAX + 2)
    def _(k):
        @pl.when((k >= 2) & (flags[jnp.maximum(k - 2, 0)] == 1))
        def _():
            scat_dma(k - 2, 0).wait()
            scat_dma(k - 2, 1).wait()

        @pl.when(k + 2 < KMAX)
        def _():
            for h in header_dma(k + 2):
                h.start()

        @pl.when(k < KMAX)
        def _():
            j = k * NSUB + sid
            start = chunk_start(k)
            slot = lax.rem(k, 4)
            for h in header_dma(k):
                h.wait()
            lo = jnp.min(bbufs[slot, pl.ds(0, 16)])
            hi = jnp.max(bbufs[slot, pl.ds(CHUNK - 16, 16)])
            rel = (hi >= glo) & (lo < glo + B // 2) & (j < NCHUNKS)
            flags[k] = rel.astype(jnp.int32)

            @pl.when(rel)
            def _():
                node_dma(k, 0).start()
                node_dma(k, 1).start()
                for g in range(CHUNK // 16):
                    b = bbufs[slot, pl.ds(g * 16, 16)]
                    c = cbufs[slot, pl.ds(g * 16, 16)]
                    sval = b * C_MAX + c - base
                    pos = start + g * 16 + iota
                    ok = (sval >= 0) & (sval < HALF) & (pos >= j * CHUNK)
                    ibufs[slot, pl.ds(g * 16, 16)] = jnp.where(ok, sval, dummy)

        @pl.when((k >= 1) & (k <= KMAX)
                 & (flags[jnp.maximum(k - 1, 0)] == 1))
        def _():
            node_dma(k - 1, 0).wait()
            node_dma(k - 1, 1).wait()
            scat_dma(k - 1, 0).start(add=True)
            scat_dma(k - 1, 1).start(add=True)

    plsc.subcore_barrier()
    row = sid * ZROWS
    pltpu.sync_copy(acc_lo.at[pl.ds(row, ZROWS)],
                    e_out.at[pl.ds(base + row, ZROWS), pl.ds(0, HD)])
    pltpu.sync_copy(acc_hi.at[pl.ds(row, ZROWS)],
                    e_out.at[pl.ds(base + row, ZROWS), pl.ds(HD, HD)])


_sc_compiler_params = pltpu.CompilerParams()
if "needs_layout_passes" in pltpu.CompilerParams.__dataclass_fields__:
    _sc_compiler_params = dataclasses.replace(
        _sc_compiler_params, needs_layout_passes=False)

_sc_segment_sum = functools.partial(
    pl.kernel,
    compiler_params=_sc_compiler_params,
    out_type=jax.ShapeDtypeStruct((NBUCKET, D), jnp.float32),
    mesh=plsc.VectorSubcoreMesh(core_axis_name="c", subcore_axis_name="s"),
    scratch_types=[
        pltpu.VMEM((2, CHUNK, D // 2), jnp.float32),
        pltpu.VMEM((2, CHUNK, D // 2), jnp.float32),
        pltpu.VMEM((4, CHUNK), jnp.int32),
        pltpu.VMEM((4, CHUNK), jnp.int32),
        pltpu.VMEM((4, CHUNK), jnp.int32),
        pltpu.SMEM((KMAX,), jnp.int32),
        pltpu.VMEM_SHARED((HALF + NSUB, D // 2), jnp.float32),
        pltpu.VMEM_SHARED((HALF + NSUB, D // 2), jnp.float32),
        pltpu.SemaphoreType.DMA,
        pltpu.SemaphoreType.DMA,
        pltpu.SemaphoreType.DMA,
        pltpu.SemaphoreType.DMA,
        pltpu.SemaphoreType.DMA,
        pltpu.SemaphoreType.DMA,
    ],
)(_sc_body)


NC_ROWS = 8
CHK = 128


def _numc_body(b_ref, c_ref, o_ref, mx_ref):
    i = pl.program_id(0)

    @pl.when(i == 0)
    def _():
        mx_ref[...] = jnp.full((1, B), -1, jnp.int32)

    giota = lax.broadcasted_iota(jnp.int32, (1, B), 1)
    bt = jnp.transpose(b_ref[...])
    ct = jnp.transpose(c_ref[...])
    mx = mx_ref[...]
    for j in range(NC_ROWS):
        cand = jnp.where(bt[:, j:j + 1] == giota,
                         ct[:, j:j + 1], -1)
        mx = jnp.maximum(mx, jnp.max(cand, axis=0, keepdims=True))
    mx_ref[...] = mx

    @pl.when(i == NPAD // (NC_ROWS * CHK) - 1)
    def _():
        o_ref[...] = (mx_ref[...] + 1).astype(jnp.float32)


def _mlp_body(e_ref, m_ref, w1_ref, b1_ref, w2_ref, b2_ref, o_ref):
    e = e_ref[...]
    h = jnp.dot(e, w1_ref[...], preferred_element_type=jnp.float32)
    h = h + b1_ref[...]
    h = jnp.where(h >= 0, h, 0.01 * h)
    val = jnp.dot(h, w2_ref[...],
                  preferred_element_type=jnp.float32) + b2_ref[0, 0]
    valm = val.reshape(-1, C_MAX)
    ciota = lax.broadcasted_iota(jnp.int32, (1, C_MAX), 1).astype(jnp.float32)
    msk = (ciota < m_ref[...]).astype(jnp.float32)
    v = jnp.sum(valm * msk, axis=1, keepdims=True)
    o_ref[...] = v


def kernel(node_embed, batch, component, W1, b1, W2, b2):
    bpad = jnp.full((NPAD - N,), B, jnp.int32)
    b2d = jnp.concatenate([batch, bpad]).reshape(NPAD // CHK, CHK)
    c2d = jnp.concatenate(
        [component, jnp.zeros((NPAD - N,), jnp.int32)]
    ).reshape(NPAD // CHK, CHK)
    numc = pl.pallas_call(
        _numc_body,
        grid=(NPAD // (NC_ROWS * CHK),),
        in_specs=[
            pl.BlockSpec((NC_ROWS, CHK), lambda i: (i, 0)),
            pl.BlockSpec((NC_ROWS, CHK), lambda i: (i, 0)),
        ],
        out_specs=pl.BlockSpec((1, B), lambda i: (0, 0)),
        out_shape=jax.ShapeDtypeStruct((1, B), jnp.float32),
        scratch_shapes=[pltpu.VMEM((1, B), jnp.int32)],
    )(b2d, c2d)
    numc = numc.reshape(B, 1)

    zeros = jnp.zeros((ZROWS, D // 2), jnp.float32)
    e = _sc_segment_sum(node_embed, batch, component, zeros)

    rows = 2048
    v = pl.pallas_call(
        _mlp_body,
        grid=(NBUCKET // rows,),
        in_specs=[
            pl.BlockSpec((rows, D), lambda i: (i, 0)),
            pl.BlockSpec((rows // C_MAX, 1), lambda i: (i, 0)),
            pl.BlockSpec((D, D), lambda i: (0, 0)),
            pl.BlockSpec((1, D), lambda i: (0, 0)),
            pl.BlockSpec((D, 1), lambda i: (0, 0)),
            pl.BlockSpec((1, 1), lambda i: (0, 0)),
        ],
        out_specs=pl.BlockSpec((rows // C_MAX, 1), lambda i: (i, 0)),
        out_shape=jax.ShapeDtypeStruct((B, 1), jnp.float32),
    )(e, numc, W1, b1.reshape(1, D), W2, b2.reshape(1, 1))
    return v
